# baseline (speedup 1.0000x reference)
import functools

import jax
import jax.numpy as jnp
from jax import lax
from jax.experimental import pallas as pl
from jax.experimental.pallas import tpu as pltpu
from jax.experimental.pallas import tpu_sc as plsc

_D = 64
_CHUNK = 128
_NC = 2
_NS = 16
_NW = _NC * _NS


def _rsqrt16(v):
    i = plsc.bitcast(v, jnp.int32)
    i = jnp.int32(0x5F3759DF) - lax.shift_right_logical(i, 1)
    y = plsc.bitcast(i, jnp.float32)
    for _ in range(3):
        y = y * (1.5 - 0.5 * v * y * y)
    return y


def _build(nchunks, seq_len):
    mesh = plsc.VectorSubcoreMesh(core_axis_name="c", subcore_axis_name="s")

    @functools.partial(
        pl.kernel,
        mesh=mesh,
        compiler_params=pltpu.CompilerParams(
            needs_layout_passes=False, use_tc_tiling_on_sc=False),
        out_type=jax.ShapeDtypeStruct((_NW, nchunks, _CHUNK, _D), jnp.float32),
        scratch_types=[
            pltpu.VMEM((nchunks, _CHUNK), jnp.int32),
            pltpu.VMEM((2 * seq_len, _D), jnp.float32),
            pltpu.VMEM((_D,), jnp.float32),
            pltpu.VMEM((_D,), jnp.float32),
            pltpu.VMEM((_CHUNK, _D), jnp.float32),
            pltpu.VMEM((_CHUNK, _D), jnp.float32),
            pltpu.SemaphoreType.DMA,
            pltpu.SemaphoreType.DMA,
        ],
    )
    def k(idx_hbm, table_hbm, pos_hbm, g_hbm, b_hbm, out_hbm,
          idx_v, pos_v, g_v, b_v, buf0, buf1, sem0, sem1):
        wid = lax.axis_index("s") * _NC + lax.axis_index("c")
        pltpu.sync_copy(idx_hbm.at[wid], idx_v)
        pltpu.sync_copy(pos_hbm, pos_v.at[pl.ds(0, seq_len)])
        pltpu.sync_copy(pos_hbm, pos_v.at[pl.ds(seq_len, seq_len)])
        pltpu.sync_copy(g_hbm, g_v)
        pltpu.sync_copy(b_hbm, b_v)
        gk = [g_v[pl.ds(t * 16, 16)] for t in range(4)]
        bk = [b_v[pl.ds(t * 16, 16)] for t in range(4)]

        bufs = (buf0, buf1)
        sems = (sem0, sem1)

        def issue(j, b):
            pltpu.async_copy(table_hbm.at[idx_v.at[j]], bufs[b], sems[b])

        def wait(j, b):
            pltpu.make_async_copy(
                table_hbm.at[idx_v.at[j]], bufs[b], sems[b]).wait()

        issue(0, 0)

        def do_chunk(j, b):
            @pl.when(j + 1 < nchunks)
            def _():
                issue(j + 1, 1 - b)
            wait(j, b)
            buf = bufs[b]
            pb = lax.rem(j * _CHUNK, seq_len)

            def row(i, c):
                ys = []
                for t in range(4):
                    x = buf[i, pl.ds(t * 16, 16)]
                    p = pos_v[pb + i, pl.ds(t * 16, 16)]
                    ys.append(x + p)
                s = (ys[0] + ys[1]) + (ys[2] + ys[3])
                mean = jnp.sum(s) * (1.0 / _D)
                d = [y - mean for y in ys]
                sq = (d[0] * d[0] + d[1] * d[1]) + (d[2] * d[2] + d[3] * d[3])
                var = jnp.sum(sq) * (1.0 / _D)
                r = _rsqrt16(jnp.broadcast_to(var + 1e-5, (16,)))
                for t in range(4):
                    buf[i, pl.ds(t * 16, 16)] = d[t] * r * gk[t] + bk[t]
                return c

            lax.fori_loop(0, _CHUNK, row, 0)
            pltpu.sync_copy(buf, out_hbm.at[wid, j])

        def outer(t, c):
            do_chunk(2 * t, 0)
            do_chunk(2 * t + 1, 1)
            return c

        lax.fori_loop(0, nchunks // 2, outer, 0)

    return k


def kernel(token_ids, token_table, pos_table, gamma, beta):
    B, L = token_ids.shape
    V, D = token_table.shape
    assert D == _D and pos_table.shape == (L, D)
    total = B * L
    assert total % (_NW * _CHUNK) == 0
    nchunks = total // (_NW * _CHUNK)
    idx3 = token_ids.astype(jnp.int32).reshape(_NW, nchunks, _CHUNK)
    out = _build(nchunks, L)(
        idx3,
        token_table.astype(jnp.float32),
        pos_table.astype(jnp.float32),
        gamma.astype(jnp.float32),
        beta.astype(jnp.float32),
    )
    return out.reshape(B, L, D)


# speedup vs baseline: 1.5786x; 1.5786x over previous
import functools

import jax
import jax.numpy as jnp
from jax import lax
from jax.experimental import pallas as pl
from jax.experimental.pallas import tpu as pltpu
from jax.experimental.pallas import tpu_sc as plsc

_D = 64
_CHUNK = 128
_NC = 2
_NS = 16
_NW = _NC * _NS


def _rsqrt16(v):
    i = plsc.bitcast(v, jnp.int32)
    i = jnp.int32(0x5F3759DF) - lax.shift_right_logical(i, 1)
    y = plsc.bitcast(i, jnp.float32)
    for _ in range(2):
        y = y * (1.5 - 0.5 * v * y * y)
    return y


def _build(nchunks, seq_len):
    mesh = plsc.VectorSubcoreMesh(core_axis_name="c", subcore_axis_name="s")

    @functools.partial(
        pl.kernel,
        mesh=mesh,
        compiler_params=pltpu.CompilerParams(
            needs_layout_passes=False, use_tc_tiling_on_sc=False),
        out_type=jax.ShapeDtypeStruct((_NW, nchunks, _CHUNK, _D), jnp.float32),
        scratch_types=[
            pltpu.VMEM((nchunks, _CHUNK), jnp.int32),
            pltpu.VMEM((_D,), jnp.float32),
            pltpu.VMEM((_D,), jnp.float32),
            pltpu.VMEM((_CHUNK, _D), jnp.float32),
            pltpu.VMEM((_CHUNK, _D), jnp.float32),
            pltpu.VMEM((_CHUNK, _D), jnp.float32),
            pltpu.VMEM((_CHUNK, _D), jnp.float32),
            pltpu.SemaphoreType.DMA,
            pltpu.SemaphoreType.DMA,
            pltpu.SemaphoreType.DMA,
            pltpu.SemaphoreType.DMA,
            pltpu.SemaphoreType.DMA,
            pltpu.SemaphoreType.DMA,
        ],
    )
    def k(idx_hbm, table_hbm, pos2_hbm, g_hbm, b_hbm, out_hbm,
          idx_v, g_v, b_v, ibuf0, ibuf1, obuf0, obuf1,
          gsem0, gsem1, ssem0, ssem1, psem0, psem1):
        wid = lax.axis_index("s") * _NC + lax.axis_index("c")
        pltpu.sync_copy(idx_hbm.at[wid], idx_v)
        pltpu.sync_copy(g_hbm, g_v)
        pltpu.sync_copy(b_hbm, b_v)
        gk = [g_v[pl.ds(t * 16, 16)] for t in range(4)]
        bk = [b_v[pl.ds(t * 16, 16)] for t in range(4)]

        ibufs = (ibuf0, ibuf1)
        obufs = (obuf0, obuf1)
        gsems = (gsem0, gsem1)
        ssems = (ssem0, ssem1)
        psems = (psem0, psem1)

        def pos_src(j):
            pb = lax.rem(j * _CHUNK, seq_len)
            return pos2_hbm.at[pl.ds(pb, _CHUNK)]

        def start_prefill(j, b):
            pltpu.async_copy(pos_src(j), ibufs[b], psems[b])

        def issue_gather(j, b):
            pltpu.make_async_copy(pos_src(j), ibufs[b], psems[b]).wait()
            pltpu.async_copy(
                table_hbm.at[idx_v.at[j]], ibufs[b], gsems[b], add=True)

        def wait_gather(j, b):
            pltpu.make_async_copy(
                table_hbm.at[idx_v.at[j]], ibufs[b], gsems[b]).wait()

        def wait_scatter(j, b):
            pltpu.make_async_copy(
                obufs[b], out_hbm.at[wid, j], ssems[b]).wait()

        start_prefill(0, 0)
        issue_gather(0, 0)

        def do_chunk(j, b):
            @pl.when(j + 1 < nchunks)
            def _():
                start_prefill(j + 1, 1 - b)
            wait_gather(j, b)

            @pl.when(j + 1 < nchunks)
            def _():
                issue_gather(j + 1, 1 - b)

            @pl.when(j >= 2)
            def _():
                wait_scatter(j - 2, b)

            ibuf = ibufs[b]
            obuf = obufs[b]

            @plsc.parallel_loop(0, _CHUNK, 1, unroll=8)
            def row(i):
                y = [ibuf[i, pl.ds(t * 16, 16)] for t in range(4)]
                s = (y[0] + y[1]) + (y[2] + y[3])
                q = (y[0] * y[0] + y[1] * y[1]) + (y[2] * y[2] + y[3] * y[3])
                mean = jnp.sum(s) * (1.0 / _D)
                msq = jnp.sum(q) * (1.0 / _D)
                var = msq - mean * mean + 1e-5
                r = _rsqrt16(jnp.broadcast_to(var, (16,)))
                for t in range(4):
                    rg = r * gk[t]
                    obuf[i, pl.ds(t * 16, 16)] = y[t] * rg - (mean * rg - bk[t])

            pltpu.async_copy(obuf, out_hbm.at[wid, j], ssems[b])

        def outer(t, c):
            do_chunk(2 * t, 0)
            do_chunk(2 * t + 1, 1)
            return c

        lax.fori_loop(0, nchunks // 2, outer, 0)
        wait_scatter(nchunks - 2, 0)
        wait_scatter(nchunks - 1, 1)

    return k


def kernel(token_ids, token_table, pos_table, gamma, beta):
    B, L = token_ids.shape
    V, D = token_table.shape
    assert D == _D and pos_table.shape == (L, D)
    total = B * L
    assert total % (_NW * _CHUNK) == 0
    nchunks = total // (_NW * _CHUNK)
    idx3 = token_ids.astype(jnp.int32).reshape(_NW, nchunks, _CHUNK)
    pos2 = jnp.concatenate([pos_table, pos_table], axis=0).astype(jnp.float32)
    out = _build(nchunks, L)(
        idx3,
        token_table.astype(jnp.float32),
        pos2,
        gamma.astype(jnp.float32),
        beta.astype(jnp.float32),
    )
    return out.reshape(B, L, D)
